# baseline (speedup 1.0000x reference)
import jax
import jax.numpy as jnp
import numpy as np
from jax.experimental import pallas as pl
from jax.experimental.pallas import tpu as pltpu

_N = 512
_T = 96
_B = 4

_HI = jax.lax.Precision.HIGHEST


def _dot_bf16(a, b, dims):
    return jax.lax.dot_general(
        a.astype(jnp.bfloat16), b.astype(jnp.bfloat16), dims,
        preferred_element_type=jnp.float32)


def _body(x_ref, W_ref, bcol_ref, t_hbm, out_ref, t_vmem, t_sem):
    f32 = jnp.float32
    bf16 = jnp.bfloat16
    t_cp = pltpu.make_async_copy(t_hbm, t_vmem, t_sem)
    t_cp.start()

    x = x_ref[...]
    W = W_ref[...]

    x_avg = (x[0] + x[1] + x[2] + x[3]) * 0.25
    sq = jnp.sum(x_avg * x_avg, axis=0)
    nrm = jnp.maximum(jnp.sqrt(sq), 1e-12)
    xn = x_avg / nrm[None, :]
    d = _dot_bf16(xn, xn, (((0,), (0,)), ((), ())))
    xw_all = jnp.concatenate(
        [_dot_bf16(x[bi], W, (((0,), (0,)), ((), ()))) for bi in range(_B)],
        axis=1)

    t_cp.wait()
    t = t_vmem[...]
    edge = (d + 1.0) >= t
    adj_b = edge.astype(bf16)

    ri = jax.lax.broadcasted_iota(jnp.int32, (_N, _N), 0)
    ci = jax.lax.broadcasted_iota(jnp.int32, (_N, _N), 1)
    nb = jnp.where(ri == ci, jnp.asarray(1.0, bf16),
                   jnp.minimum(adj_b + adj_b.T, jnp.asarray(1.0, bf16)))
    common = jax.lax.dot_general(
        nb, nb, (((1,), (1,)), ((), ())),
        preferred_element_type=f32)
    s = jnp.where(edge & (common > 1.0), common * common, 0.0)
    S = jnp.sum(s, axis=0)
    r_inv = jnp.where(S > 0.0, jax.lax.rsqrt(S), 0.0)

    xw_s = xw_all * r_inv[:, None]
    y_all = _dot_bf16(xw_s, s, (((0,), (0,)), ((), ())))
    bcol = bcol_ref[...]
    for bi in range(_B):
        out_ref[bi] = y_all[bi * _T:(bi + 1) * _T, :] * r_inv[None, :] + bcol


def _threefry2x32(k1, k2, x1, x2):
    def rotl(v, r):
        return ((v << np.uint32(r)) | (v >> np.uint32(32 - r))) & np.uint32(0xFFFFFFFF)

    ks = [k1, k2, k1 ^ k2 ^ np.uint32(0x1BD11BDA)]
    rot_a = (13, 15, 26, 6)
    rot_b = (17, 29, 16, 24)
    x1 = x1 + ks[0]
    x2 = x2 + ks[1]
    for j, rots in enumerate((rot_a, rot_b, rot_a, rot_b, rot_a)):
        for r in rots:
            x1 = x1 + x2
            x2 = rotl(x2, r)
            x2 = x2 ^ x1
        x1 = x1 + ks[(j + 1) % 3]
        x2 = x2 + ks[(j + 2) % 3] + np.uint32(j + 1)
    return x1, x2


def _gumbel_consts():
    n = _N * _N
    with np.errstate(over="ignore"):
        idx = np.arange(2 * n, dtype=np.uint32)
        b1, b2 = _threefry2x32(np.uint32(0), np.uint32(42),
                               np.zeros(2 * n, dtype=np.uint32), idx)
    bits = b1 ^ b2
    f = ((bits >> np.uint32(9)) | np.uint32(0x3F800000)).view(np.float32)
    one, minv = np.float32(1.0), np.float32(1e-20)
    u = np.maximum(minv, (f - one) * (one - minv) + minv)
    g = -np.log(-np.log(u))
    g = g.reshape(n, 2)
    g0 = g[:, 0].reshape(_N, _N).astype(np.float64)
    g1 = g[:, 1].reshape(_N, _N).astype(np.float64)
    return np.ascontiguousarray((1.0 + g1 - g0).astype(np.float32))


_THRESH = _gumbel_consts()


def kernel(x, W, b):
    bcol = b.reshape(_T, 1)
    return pl.pallas_call(
        _body,
        in_specs=[
            pl.BlockSpec(memory_space=pltpu.VMEM),
            pl.BlockSpec(memory_space=pltpu.VMEM),
            pl.BlockSpec(memory_space=pltpu.VMEM),
            pl.BlockSpec(memory_space=pl.ANY),
        ],
        out_specs=pl.BlockSpec(memory_space=pltpu.VMEM),
        scratch_shapes=[
            pltpu.VMEM((_N, _N), jnp.float32),
            pltpu.SemaphoreType.DMA,
        ],
        out_shape=jax.ShapeDtypeStruct((_B, _T, _N), jnp.float32),
    )(x, W, bcol, _THRESH)


# speedup vs baseline: 1.1256x; 1.1256x over previous
import jax
import jax.numpy as jnp
import numpy as np
from jax.experimental import pallas as pl
from jax.experimental.pallas import tpu as pltpu

_N = 512
_T = 96
_B = 4

_HI = jax.lax.Precision.HIGHEST


def _dot_bf16(a, b, dims):
    return jax.lax.dot_general(
        a.astype(jnp.bfloat16), b.astype(jnp.bfloat16), dims,
        preferred_element_type=jnp.float32)


def _body(x_ref, W_ref, bcol_ref, t_ref, out_ref):
    f32 = jnp.float32
    bf16 = jnp.bfloat16
    x = x_ref[...]
    W = W_ref[...]

    x_avg = (x[0] + x[1] + x[2] + x[3]) * 0.25
    sq = jnp.sum(x_avg * x_avg, axis=0)
    nrm = jnp.maximum(jnp.sqrt(sq), 1e-12)
    xn = x_avg / nrm[None, :]
    d = _dot_bf16(xn, xn, (((0,), (0,)), ((), ())))
    xw_all = jnp.concatenate(
        [_dot_bf16(x[bi], W, (((0,), (0,)), ((), ()))) for bi in range(_B)],
        axis=1)

    t = t_ref[...]
    edge = (d + 1.0) >= t
    adj_b = edge.astype(bf16)

    ri = jax.lax.broadcasted_iota(jnp.int32, (_N, _N), 0)
    ci = jax.lax.broadcasted_iota(jnp.int32, (_N, _N), 1)
    nb = jnp.where(ri == ci, jnp.asarray(1.0, bf16),
                   jnp.minimum(adj_b + adj_b.T, jnp.asarray(1.0, bf16)))
    common = jax.lax.dot_general(
        nb, nb, (((1,), (1,)), ((), ())),
        preferred_element_type=f32)
    s = jnp.where(edge & (common > 1.0), common * common, 0.0)
    S = jnp.sum(s, axis=0)
    r_inv = jnp.where(S > 0.0, jax.lax.rsqrt(S), 0.0)

    xw_s = xw_all * r_inv[:, None]
    y_all = _dot_bf16(xw_s, s, (((0,), (0,)), ((), ())))
    bcol = bcol_ref[...]
    for bi in range(_B):
        out_ref[bi] = y_all[bi * _T:(bi + 1) * _T, :] * r_inv[None, :] + bcol


def _threefry2x32(k1, k2, x1, x2):
    def rotl(v, r):
        return ((v << np.uint32(r)) | (v >> np.uint32(32 - r))) & np.uint32(0xFFFFFFFF)

    ks = [k1, k2, k1 ^ k2 ^ np.uint32(0x1BD11BDA)]
    rot_a = (13, 15, 26, 6)
    rot_b = (17, 29, 16, 24)
    x1 = x1 + ks[0]
    x2 = x2 + ks[1]
    for j, rots in enumerate((rot_a, rot_b, rot_a, rot_b, rot_a)):
        for r in rots:
            x1 = x1 + x2
            x2 = rotl(x2, r)
            x2 = x2 ^ x1
        x1 = x1 + ks[(j + 1) % 3]
        x2 = x2 + ks[(j + 2) % 3] + np.uint32(j + 1)
    return x1, x2


def _gumbel_consts():
    n = _N * _N
    with np.errstate(over="ignore"):
        idx = np.arange(2 * n, dtype=np.uint32)
        b1, b2 = _threefry2x32(np.uint32(0), np.uint32(42),
                               np.zeros(2 * n, dtype=np.uint32), idx)
    bits = b1 ^ b2
    f = ((bits >> np.uint32(9)) | np.uint32(0x3F800000)).view(np.float32)
    one, minv = np.float32(1.0), np.float32(1e-20)
    u = np.maximum(minv, (f - one) * (one - minv) + minv)
    g = -np.log(-np.log(u))
    g = g.reshape(n, 2)
    g0 = g[:, 0].reshape(_N, _N).astype(np.float64)
    g1 = g[:, 1].reshape(_N, _N).astype(np.float64)
    return np.ascontiguousarray((1.0 + g1 - g0).astype(np.float32))


_THRESH = _gumbel_consts()


def kernel(x, W, b):
    bcol = b.reshape(_T, 1)
    return pl.pallas_call(
        _body,
        out_shape=jax.ShapeDtypeStruct((_B, _T, _N), jnp.float32),
    )(x, W, bcol, _THRESH)
